# baseline (speedup 1.0000x reference)
import jax
import jax.numpy as jnp
from jax.experimental import pallas as pl

_N = 4096
_K = 30
_NEG = -1e15
_ROWS_PER_BLOCK = 256


def _topk_softmax_kernel(x_ref, o_ref):
    x = x_ref[...]
    bi = jax.lax.bitcast_convert_type(x, jnp.int32)
    key = bi ^ jnp.bitwise_and(jnp.right_shift(bi, 31), jnp.int32(0x7FFFFFFF))
    min32 = jnp.int32(-(2**31))
    h = _ROWS_PER_BLOCK // 2
    ka, kb = key[:h], key[h:]

    def body(i, ws):
        wa, wb = ws
        bit = jnp.left_shift(jnp.int32(1), jnp.int32(31) - i)
        ca, cb = jnp.bitwise_or(wa, bit), jnp.bitwise_or(wb, bit)
        ta = jnp.bitwise_xor(ca, min32)
        tb = jnp.bitwise_xor(cb, min32)
        na = jnp.sum((ka >= ta).astype(jnp.float32), axis=1, keepdims=True)
        nb = jnp.sum((kb >= tb).astype(jnp.float32), axis=1, keepdims=True)
        return (jnp.where(na >= _K, ca, wa), jnp.where(nb >= _K, cb, wb))

    w0 = jnp.zeros((h, 1), jnp.int32)
    wa, wb = jax.lax.fori_loop(0, 32, body, (w0, w0), unroll=32)
    kth = jnp.bitwise_xor(jnp.concatenate([wa, wb], axis=0), min32)

    keep = (key >= kth) & (x > 0.0)
    m = jnp.where(keep, x, _NEG)
    rowmax = jnp.max(m, axis=1, keepdims=True)
    e = jnp.exp(m - rowmax)
    s = jnp.sum(e, axis=1, keepdims=True)
    o_ref[...] = e / s


def kernel(new_supports):
    n = new_supports.shape[0]
    r = _ROWS_PER_BLOCK
    return pl.pallas_call(
        _topk_softmax_kernel,
        grid=(n // r,),
        in_specs=[pl.BlockSpec((r, _N), lambda i: (i, 0))],
        out_specs=pl.BlockSpec((r, _N), lambda i: (i, 0)),
        out_shape=jax.ShapeDtypeStruct((n, _N), jnp.float32),
    )(new_supports)
